# baseline (speedup 1.0000x reference)
import functools

import jax
import jax.numpy as jnp
from jax import lax
from jax.experimental import pallas as pl
from jax.experimental.pallas import tpu as pltpu
from jax.experimental.pallas import tpu_sc as plsc

B = 8192
B2 = 2 * B
EMBED = 16
HIDDEN = 64

NUM_CORES = 2
NUM_SUBCORES = 16
NW = NUM_CORES * NUM_SUBCORES
PER_W = B2 // NW
CHUNK = 128
NCHUNK = PER_W // CHUNK


def _gather_body(uid_hbm, iid_hbm, uemb_hbm, iemb_hbm, uout_hbm, vout_hbm,
                 uidx_v, iidx_v, urows_v, vrows_v, sem):
    wid = lax.axis_index("s") * NUM_CORES + lax.axis_index("c")
    pltpu.sync_copy(uid_hbm.at[wid], uidx_v)
    pltpu.sync_copy(iid_hbm.at[wid], iidx_v)
    copies = []
    for j in range(NCHUNK):
        copies.append(pltpu.make_async_copy(
            uemb_hbm.at[uidx_v.at[j]], urows_v.at[pl.ds(j * CHUNK, CHUNK)], sem))
        copies.append(pltpu.make_async_copy(
            iemb_hbm.at[iidx_v.at[j]], vrows_v.at[pl.ds(j * CHUNK, CHUNK)], sem))
    for c in copies:
        c.start()
    for c in copies:
        c.wait()
    pltpu.sync_copy(urows_v, uout_hbm.at[wid])
    pltpu.sync_copy(vrows_v, vout_hbm.at[wid])


@functools.partial(jax.jit, static_argnums=())
def _sc_gather(uid_r, iid_r, user_emb, item_emb):
    mesh = plsc.VectorSubcoreMesh(core_axis_name="c", subcore_axis_name="s")
    return pl.kernel(
        _gather_body,
        mesh=mesh,
        out_type=(
            jax.ShapeDtypeStruct((NW, PER_W, EMBED), jnp.float32),
            jax.ShapeDtypeStruct((NW, PER_W, EMBED), jnp.float32),
        ),
        scratch_types=[
            pltpu.VMEM((NCHUNK, CHUNK), jnp.int32),
            pltpu.VMEM((NCHUNK, CHUNK), jnp.int32),
            pltpu.VMEM((PER_W, EMBED), jnp.float32),
            pltpu.VMEM((PER_W, EMBED), jnp.float32),
            pltpu.SemaphoreType.DMA,
        ],
        compiler_params=pltpu.CompilerParams(use_tc_tiling_on_sc=False),
    )(uid_r, iid_r, user_emb, item_emb)


ROWS_BLK = 2048
NBLK = B2 // ROWS_BLK


def _mlp_body(u_ref, v_ref, yf_ref, w1t_ref, b1_ref, w2_ref, b2_ref,
              pred_ref, loss_ref):
    i = pl.program_id(0)
    u = jnp.maximum(u_ref[...], 0.0)
    v = jnp.maximum(v_ref[...], 0.0)
    w1t = w1t_ref[...]
    h = (jnp.dot(u, w1t[:EMBED, :], preferred_element_type=jnp.float32)
         + jnp.dot(v, w1t[EMBED:, :], preferred_element_type=jnp.float32)
         + b1_ref[...])
    h = jnp.maximum(h, 0.0)
    z = jnp.sum(h * w2_ref[...], axis=1) + b2_ref[0, 0]
    p = jax.nn.sigmoid(z)
    pred_ref[0, 0, :] = p
    t = yf_ref[0, 0, :]
    log_p = jnp.maximum(jnp.log(p), -100.0)
    log_1mp = jnp.maximum(jnp.log(1.0 - p), -100.0)
    part = jnp.sum(t * log_p + (1.0 - t) * log_1mp)

    @pl.when(i == 0)
    def _init():
        loss_ref[0, 0] = 0.0

    loss_ref[0, 0] += part


def _tc_mlp(u, v, y_full, W1T, b1, W2, b2):
    pred, loss_sum = pl.pallas_call(
        _mlp_body,
        grid=(NBLK,),
        in_specs=[
            pl.BlockSpec((ROWS_BLK, EMBED), lambda i: (i, 0)),
            pl.BlockSpec((ROWS_BLK, EMBED), lambda i: (i, 0)),
            pl.BlockSpec((1, 1, ROWS_BLK), lambda i: (i, 0, 0)),
            pl.BlockSpec((2 * EMBED, HIDDEN), lambda i: (0, 0)),
            pl.BlockSpec((1, HIDDEN), lambda i: (0, 0)),
            pl.BlockSpec((1, HIDDEN), lambda i: (0, 0)),
            pl.BlockSpec((1, 1), lambda i: (0, 0)),
        ],
        out_specs=[
            pl.BlockSpec((1, 1, ROWS_BLK), lambda i: (i, 0, 0)),
            pl.BlockSpec(memory_space=pltpu.SMEM),
        ],
        out_shape=[
            jax.ShapeDtypeStruct((NBLK, 1, ROWS_BLK), jnp.float32),
            jax.ShapeDtypeStruct((1, 1), jnp.float32),
        ],
    )(u, v, y_full, W1T, b1, W2, b2)
    return pred, loss_sum


def kernel(y, uid, iid, user_emb, item_emb, W1, b1, W2, b2):
    uid_r = uid.reshape(NW, NCHUNK, CHUNK)
    iid_r = iid.reshape(NW, NCHUNK, CHUNK)
    u3, v3 = _sc_gather(uid_r, iid_r, user_emb, item_emb)
    u = u3.reshape(B2, EMBED)
    v = v3.reshape(B2, EMBED)
    y_full = jnp.concatenate([y, jnp.zeros_like(y)]).reshape(NBLK, 1, ROWS_BLK)
    pred, loss_sum = _tc_mlp(u, v, y_full, W1.T, b1.reshape(1, HIDDEN),
                             W2.reshape(1, HIDDEN), b2.reshape(1, 1))
    prediction = pred.reshape(B2, 1)
    loss = -loss_sum[0, 0] / jnp.float32(B2)
    return prediction, loss


# speedup vs baseline: 5.2499x; 5.2499x over previous
import jax
import jax.numpy as jnp
from jax import lax
from jax.experimental import pallas as pl
from jax.experimental.pallas import tpu as pltpu
from jax.experimental.pallas import tpu_sc as plsc

B = 8192
B2 = 2 * B
EMBED = 16
HIDDEN = 64
NROWS = 1000000

NUM_CORES = 2
NUM_SUBCORES = 16
NW = NUM_CORES * NUM_SUBCORES
PER_W = B2 // NW
CHUNK = 128
NCHUNK = PER_W // CHUNK
L = 16

TCOL = 7813
BT = 782
NGB = (TCOL + BT - 1) // BT
TCOLP = BT * NGB
GROUP = TCOLP * 1024
FLATPAD = 2 * GROUP


def _stage_body(t_ref, out_ref):
    x = t_ref[...]
    out_ref[0, :, :, :] = jnp.swapaxes(x.reshape(8, BT, 128), 0, 1)


def _tc_stage(tt):
    out = pl.pallas_call(
        _stage_body,
        grid=(2, NGB),
        in_specs=[pl.BlockSpec((8, BT * 128), lambda a, g: (a, g))],
        out_specs=pl.BlockSpec((1, BT, 8, 128), lambda a, g: (a, g, 0, 0)),
        out_shape=jax.ShapeDtypeStruct((2, TCOLP, 8, 128), jnp.float32),
    )(tt)
    return out.reshape(FLATPAD)


def _gather_body(uid_hbm, iid_hbm, uflat_hbm, iflat_hbm, uout_hbm, vout_hbm,
                 idx_u, idx_i, fidx_u, fidx_i, urows, vrows, sem):
    wid = lax.axis_index("s") * NUM_CORES + lax.axis_index("c")
    pltpu.sync_copy(uid_hbm.at[wid], idx_u)
    pltpu.sync_copy(iid_hbm.at[wid], idx_i)
    for c in range(NCHUNK):
        for j in range(CHUNK // L):
            ru = idx_u[c, pl.ds(j * L, L)]
            ri = idx_i[c, pl.ds(j * L, L)]
            bu = (ru >> 7) * 1024 + (ru & 127)
            bi = (ri >> 7) * 1024 + (ri & 127)
            for f in range(EMBED):
                off = (f // 8) * GROUP + (f % 8) * 128
                fidx_u[c, f, pl.ds(j * L, L)] = bu + off
                fidx_i[c, f, pl.ds(j * L, L)] = bi + off
    copies = []
    for c in range(NCHUNK):
        for f in range(EMBED):
            copies.append(pltpu.make_async_copy(
                uflat_hbm.at[fidx_u.at[c, f]],
                urows.at[f, pl.ds(c * CHUNK, CHUNK)], sem))
            copies.append(pltpu.make_async_copy(
                iflat_hbm.at[fidx_i.at[c, f]],
                vrows.at[f, pl.ds(c * CHUNK, CHUNK)], sem))
    for cp in copies:
        cp.start()
    for cp in copies:
        cp.wait()
    pltpu.sync_copy(urows, uout_hbm.at[wid])
    pltpu.sync_copy(vrows, vout_hbm.at[wid])


def _sc_gather(uid_r, iid_r, uflat, iflat):
    mesh = plsc.VectorSubcoreMesh(core_axis_name="c", subcore_axis_name="s")
    return pl.kernel(
        _gather_body,
        mesh=mesh,
        out_type=(
            jax.ShapeDtypeStruct((NW, EMBED, PER_W), jnp.float32),
            jax.ShapeDtypeStruct((NW, EMBED, PER_W), jnp.float32),
        ),
        scratch_types=[
            pltpu.VMEM((NCHUNK, CHUNK), jnp.int32),
            pltpu.VMEM((NCHUNK, CHUNK), jnp.int32),
            pltpu.VMEM((NCHUNK, EMBED, CHUNK), jnp.int32),
            pltpu.VMEM((NCHUNK, EMBED, CHUNK), jnp.int32),
            pltpu.VMEM((EMBED, PER_W), jnp.float32),
            pltpu.VMEM((EMBED, PER_W), jnp.float32),
            pltpu.SemaphoreType.DMA,
        ],
        compiler_params=pltpu.CompilerParams(use_tc_tiling_on_sc=False),
    )(uid_r, iid_r, uflat, iflat)


def _mlp_body(u_ref, v_ref, yf_ref, w1_ref, b1_ref, w2_ref, b2_ref,
              pred_ref, loss_ref):
    i = pl.program_id(0)
    u = jnp.maximum(u_ref[0, :, :], 0.0)
    v = jnp.maximum(v_ref[0, :, :], 0.0)
    w1 = w1_ref[...]
    h = (jnp.dot(w1[:, :EMBED], u, preferred_element_type=jnp.float32)
         + jnp.dot(w1[:, EMBED:], v, preferred_element_type=jnp.float32)
         + b1_ref[...])
    h = jnp.maximum(h, 0.0)
    z = jnp.dot(w2_ref[...], h, preferred_element_type=jnp.float32) + b2_ref[0, 0]
    p = jax.nn.sigmoid(z)
    pred_ref[0, :, :] = p
    t = yf_ref[0, :, :]
    log_p = jnp.maximum(jnp.log(p), -100.0)
    log_1mp = jnp.maximum(jnp.log(1.0 - p), -100.0)
    part = jnp.sum(t * log_p + (1.0 - t) * log_1mp)

    @pl.when(i == 0)
    def _init():
        loss_ref[0, 0] = 0.0

    loss_ref[0, 0] += part


def _tc_mlp(u3, v3, y_full, W1, b1, W2, b2):
    pred, loss_sum = pl.pallas_call(
        _mlp_body,
        grid=(NW,),
        in_specs=[
            pl.BlockSpec((1, EMBED, PER_W), lambda i: (i, 0, 0)),
            pl.BlockSpec((1, EMBED, PER_W), lambda i: (i, 0, 0)),
            pl.BlockSpec((1, 1, PER_W), lambda i: (i, 0, 0)),
            pl.BlockSpec((HIDDEN, 2 * EMBED), lambda i: (0, 0)),
            pl.BlockSpec((HIDDEN, 1), lambda i: (0, 0)),
            pl.BlockSpec((1, HIDDEN), lambda i: (0, 0)),
            pl.BlockSpec((1, 1), lambda i: (0, 0)),
        ],
        out_specs=[
            pl.BlockSpec((1, 1, PER_W), lambda i: (i, 0, 0)),
            pl.BlockSpec(memory_space=pltpu.SMEM),
        ],
        out_shape=[
            jax.ShapeDtypeStruct((NW, 1, PER_W), jnp.float32),
            jax.ShapeDtypeStruct((1, 1), jnp.float32),
        ],
    )(u3, v3, y_full, W1, b1, W2, b2)
    return pred, loss_sum


def kernel(y, uid, iid, user_emb, item_emb, W1, b1, W2, b2):
    uid_r = uid.reshape(NW, NCHUNK, CHUNK)
    iid_r = iid.reshape(NW, NCHUNK, CHUNK)
    uflat = _tc_stage(user_emb.T)
    iflat = _tc_stage(item_emb.T)
    u3, v3 = _sc_gather(uid_r, iid_r, uflat, iflat)
    y_full = jnp.concatenate([y, jnp.zeros_like(y)]).reshape(NW, 1, PER_W)
    pred, loss_sum = _tc_mlp(u3, v3, y_full, W1, b1.reshape(HIDDEN, 1),
                             W2, b2.reshape(1, 1))
    prediction = pred.reshape(B2, 1)
    loss = -loss_sum[0, 0] / jnp.float32(B2)
    return prediction, loss
